# baseline (speedup 1.0000x reference)
import functools

import jax
import jax.numpy as jnp
from jax import lax
from jax.experimental import pallas as pl
from jax.experimental.pallas import tpu as pltpu
from jax.experimental.pallas import tpu_sc as plsc

N_V = 10000
N_E = 10000
NNZ = 320000
D = 128
DH = D // 2

NC = 2
NS = 16

CH = 128
NCHUNK = 160
NHALF = NCHUNK // 2
NBUF = 8
NNZ_PAD = NS * NCHUNK * CH - NNZ

RP = 10240
RPT = RP // NS


def _sc_v2v_pass(table2, v_idx, e_idx):
  mesh = plsc.VectorSubcoreMesh(
      core_axis_name="c", subcore_axis_name="s", num_cores=NC,
      num_subcores=NS)

  @functools.partial(
      pl.kernel,
      out_type=(
          jax.ShapeDtypeStruct((NC, RP, DH), jnp.float32),
          jax.ShapeDtypeStruct((NC, RP, DH), jnp.float32),
      ),
      mesh=mesh,
      scratch_types=[
          pltpu.VMEM((NHALF, CH), jnp.int32),
          pltpu.VMEM((NHALF, CH), jnp.int32),
          [pltpu.VMEM((CH, DH), jnp.float32) for _ in range(NBUF)],
          pltpu.VMEM((16, DH), jnp.float32),
          pltpu.VMEM((RPT,), jnp.float32),
          pltpu.VMEM((CH,), jnp.float32),
          pltpu.VMEM_SHARED((RP, DH), jnp.float32),
          pltpu.VMEM_SHARED((RP,), jnp.float32),
          [pltpu.SemaphoreType.DMA for _ in range(NBUF)],
          [pltpu.SemaphoreType.DMA for _ in range(NBUF)],
          [pltpu.SemaphoreType.DMA for _ in range(NBUF)],
      ],
      compiler_params=pltpu.CompilerParams(use_tc_tiling_on_sc=False),
  )
  def v2v_kernel(table_hbm, v_hbm, e_hbm, xe_out, p_out,
                 vbuf, ebuf, rows, zrow, cntv, ones,
                 acc_sh, cnt_sh, gsem, ssem, csem):
    c = lax.axis_index("c")
    s = lax.axis_index("s")
    row0 = s * RPT

    zeros16 = jnp.zeros((16,), jnp.float32)
    ones16 = jnp.full((16,), 1.0, jnp.float32)

    def fill_zrow(i, carry):
      for j in range(DH // 16):
        zrow[i, pl.ds(j * 16, 16)] = zeros16
      return carry
    lax.fori_loop(0, 16, fill_zrow, 0)

    def fill_zcnt(i, carry):
      cntv[pl.ds(i * 16, 16)] = zeros16
      return carry

    for j in range(CH // 16):
      ones[pl.ds(j * 16, 16)] = ones16

    def zero_acc():
      for k in range(RPT // 16):
        pltpu.sync_copy(zrow, acc_sh.at[pl.ds(row0 + k * 16, 16)])
      lax.fori_loop(0, RPT // 16, fill_zcnt, 0)
      pltpu.sync_copy(cntv, cnt_sh.at[pl.ds(row0, RPT)])

    zero_acc()
    plsc.subcore_barrier()

    def run_stage(src_table, gather_by_v):
      gidx, sidx = (vbuf, ebuf) if gather_by_v else (ebuf, vbuf)

      def g_start(k, j):
        pltpu.async_copy(src_table.at[c].at[gidx.at[j]], rows[k], gsem[k])

      def g_wait(k):
        pltpu.make_async_copy(src_table.at[c].at[gidx.at[0]], rows[k],
                              gsem[k]).wait()

      def s_start(k, j):
        pltpu.async_copy(rows[k], acc_sh.at[sidx.at[j]], ssem[k], add=True)

      def s_wait(k):
        pltpu.make_async_copy(rows[k], acc_sh.at[sidx.at[0]],
                              ssem[k]).wait()

      def c_wait(k):
        pltpu.make_async_copy(ones, cnt_sh.at[sidx.at[0]], csem[k]).wait()

      for h in range(2):
        pltpu.sync_copy(v_hbm.at[s, pl.ds(h * NHALF, NHALF)], vbuf)
        pltpu.sync_copy(e_hbm.at[s, pl.ds(h * NHALF, NHALF)], ebuf)

        for k in range(NBUF):
          g_start(k, k)

        def body(t, carry):
          j0 = t * NBUF
          for k in range(NBUF):
            j = j0 + k
            g_wait(k)
            s_start(k, j)
            pltpu.async_copy(ones, cnt_sh.at[sidx.at[j]], csem[k], add=True)
          for k in range(NBUF):
            j2 = j0 + k + NBUF
            s_wait(k)
            c_wait(k)
            @pl.when(j2 < NHALF)
            def _():
              g_start(k, j2)
          return carry
        lax.fori_loop(0, NHALF // NBUF, body, 0)

    def divide_writeout(dst, relu):
      pltpu.sync_copy(cnt_sh.at[pl.ds(row0, RPT)], cntv)
      for k in range(RPT // CH):
        pltpu.sync_copy(acc_sh.at[pl.ds(row0 + k * CH, CH)], rows[0])

        def div_group(g, carry):
          cv = cntv[pl.ds(k * CH + g * 16, 16)]
          rcpv = 1.0 / jnp.maximum(cv, 1.0)
          for rl in range(16):
            rcp = rcpv[rl]
            r = g * 16 + rl
            for q in range(DH // 16):
              x = rows[0][r, pl.ds(q * 16, 16)] * rcp
              if relu:
                x = jnp.maximum(x, 0.0)
              rows[0][r, pl.ds(q * 16, 16)] = x
          return carry
        lax.fori_loop(0, CH // 16, div_group, 0)

        pltpu.sync_copy(rows[0], dst.at[c, pl.ds(row0 + k * CH, CH)])

    run_stage(table_hbm, gather_by_v=True)
    plsc.subcore_barrier()
    divide_writeout(xe_out, relu=False)
    zero_acc()
    plsc.subcore_barrier()

    run_stage(xe_out, gather_by_v=False)
    plsc.subcore_barrier()
    divide_writeout(p_out, relu=True)

  return v2v_kernel(table2, v_idx, e_idx)


def _matmul_relu(X, WT, b2):
  blk = 1000

  def mm_body(x_ref, w_ref, b_ref, o_ref):
    h = jnp.dot(x_ref[...], w_ref[...], preferred_element_type=jnp.float32)
    h = jnp.maximum(h + b_ref[...], 0.0)
    o_ref[0] = h[:, :DH]
    o_ref[1] = h[:, DH:]

  return pl.pallas_call(
      mm_body,
      grid=(N_V // blk,),
      in_specs=[
          pl.BlockSpec((blk, D), lambda i: (i, 0)),
          pl.BlockSpec((D, D), lambda i: (0, 0)),
          pl.BlockSpec((1, D), lambda i: (0, 0)),
      ],
      out_specs=pl.BlockSpec((NC, blk, DH), lambda i: (0, i, 0)),
      out_shape=jax.ShapeDtypeStruct((NC, RP, DH), jnp.float32),
  )(X, WT, b2)


def kernel(X, W, b, edge_index, drop_rate):
  pad = N_V + (jnp.arange(NNZ_PAD, dtype=jnp.int32) % (RP - N_V))
  v = jnp.concatenate([edge_index[0].astype(jnp.int32), pad])
  e = jnp.concatenate([edge_index[1].astype(jnp.int32), pad])
  v = v.reshape(NS, NCHUNK, CH)
  e = e.reshape(NS, NCHUNK, CH)

  H2 = _matmul_relu(X, W.T, b.reshape(1, D))

  _, P2 = _sc_v2v_pass(H2, v, e)

  return jnp.concatenate([P2[0], P2[1]], axis=-1)[:N_V]


# speedup vs baseline: 1.0786x; 1.0786x over previous
import functools

import jax
import jax.numpy as jnp
from jax import lax
from jax.experimental import pallas as pl
from jax.experimental.pallas import tpu as pltpu
from jax.experimental.pallas import tpu_sc as plsc

N_V = 10000
N_E = 10000
NNZ = 320000
D = 128
DH = D // 2

NC = 2
NS = 16

CH = 128
NCHUNK = 160
NBUF = 5
NNZ_PAD = NS * NCHUNK * CH - NNZ

RP = 10240
RPT = RP // NS


def _sc_v2v_pass(table2, v_idx, e_idx):
  mesh = plsc.VectorSubcoreMesh(
      core_axis_name="c", subcore_axis_name="s", num_cores=NC,
      num_subcores=NS)

  @functools.partial(
      pl.kernel,
      out_type=(
          jax.ShapeDtypeStruct((NC, RP, DH), jnp.float32),
          jax.ShapeDtypeStruct((RP, D), jnp.float32),
      ),
      mesh=mesh,
      scratch_types=[
          pltpu.VMEM((NCHUNK, CH), jnp.int32),
          pltpu.VMEM((NCHUNK, CH), jnp.int32),
          [pltpu.VMEM((CH, DH), jnp.float32) for _ in range(NBUF)],
          pltpu.VMEM((32, DH), jnp.float32),
          pltpu.VMEM((RPT,), jnp.float32),
          pltpu.VMEM((CH,), jnp.float32),
          pltpu.VMEM_SHARED((RP, DH), jnp.float32),
          pltpu.VMEM_SHARED((RP,), jnp.float32),
          [pltpu.SemaphoreType.DMA for _ in range(NBUF)],
          [pltpu.SemaphoreType.DMA for _ in range(NBUF)],
          [pltpu.SemaphoreType.DMA for _ in range(NBUF)],
      ],
      compiler_params=pltpu.CompilerParams(use_tc_tiling_on_sc=False),
  )
  def v2v_kernel(table_hbm, v_hbm, e_hbm, xe_out, p_out,
                 vbuf, ebuf, rows, zrow, cntv, ones,
                 acc_sh, cnt_sh, gsem, ssem, csem):
    c = lax.axis_index("c")
    s = lax.axis_index("s")
    row0 = s * RPT

    zeros16 = jnp.zeros((16,), jnp.float32)
    ones16 = jnp.full((16,), 1.0, jnp.float32)

    def fill_zrow(i, carry):
      for j in range(DH // 16):
        zrow[i, pl.ds(j * 16, 16)] = zeros16
      return carry
    lax.fori_loop(0, 32, fill_zrow, 0)

    def fill_zcnt(i, carry):
      cntv[pl.ds(i * 16, 16)] = zeros16
      return carry

    for j in range(CH // 16):
      ones[pl.ds(j * 16, 16)] = ones16

    def zero_acc():
      for k in range(RPT // 32):
        pltpu.sync_copy(zrow, acc_sh.at[pl.ds(row0 + k * 32, 32)])
      lax.fori_loop(0, RPT // 16, fill_zcnt, 0)
      pltpu.sync_copy(cntv, cnt_sh.at[pl.ds(row0, RPT)])

    zero_acc()
    plsc.subcore_barrier()

    pltpu.sync_copy(v_hbm.at[s], vbuf)
    pltpu.sync_copy(e_hbm.at[s], ebuf)

    def run_stage(src_table, gidx, sidx):
      def g_start(k, j):
        pltpu.async_copy(src_table.at[c].at[gidx.at[j]], rows[k], gsem[k])

      def g_wait(k):
        pltpu.make_async_copy(src_table.at[c].at[gidx.at[0]], rows[k],
                              gsem[k]).wait()

      def s_start(k, j):
        pltpu.async_copy(rows[k], acc_sh.at[sidx.at[j]], ssem[k], add=True)

      def s_wait(k):
        pltpu.make_async_copy(rows[k], acc_sh.at[sidx.at[0]],
                              ssem[k]).wait()

      def c_wait(k):
        pltpu.make_async_copy(ones, cnt_sh.at[sidx.at[0]], csem[k]).wait()

      for k in range(NBUF):
        g_start(k, k)

      def body(t, carry):
        j0 = t * NBUF
        for k in range(NBUF):
          j = j0 + k
          g_wait(k)
          s_start(k, j)
          pltpu.async_copy(ones, cnt_sh.at[sidx.at[j]], csem[k], add=True)
        for k in range(NBUF):
          j2 = j0 + k + NBUF
          s_wait(k)
          c_wait(k)
          @pl.when(j2 < NCHUNK)
          def _():
            g_start(k, j2)
        return carry
      lax.fori_loop(0, NCHUNK // NBUF, body, 0)

    def divide_writeout(dst_for_block, relu):
      pltpu.sync_copy(cnt_sh.at[pl.ds(row0, RPT)], cntv)
      for k in range(RPT // CH):
        pltpu.sync_copy(acc_sh.at[pl.ds(row0 + k * CH, CH)], rows[0])

        def div_group(g, carry):
          cv = cntv[pl.ds(k * CH + g * 16, 16)]
          rcpv = 1.0 / jnp.maximum(cv, 1.0)
          for rl in range(16):
            rcp = rcpv[rl]
            r = g * 16 + rl
            for q in range(DH // 16):
              x = rows[0][r, pl.ds(q * 16, 16)] * rcp
              if relu:
                x = jnp.maximum(x, 0.0)
              rows[0][r, pl.ds(q * 16, 16)] = x
          return carry
        lax.fori_loop(0, CH // 16, div_group, 0)

        pltpu.sync_copy(rows[0], dst_for_block(k))

    run_stage(table_hbm, vbuf, ebuf)
    plsc.subcore_barrier()
    divide_writeout(
        lambda k: xe_out.at[c, pl.ds(row0 + k * CH, CH)], relu=False)
    zero_acc()
    plsc.subcore_barrier()

    run_stage(xe_out, ebuf, vbuf)
    plsc.subcore_barrier()
    divide_writeout(
        lambda k: p_out.at[pl.ds(row0 + k * CH, CH), pl.ds(c * DH, DH)],
        relu=True)

  return v2v_kernel(table2, v_idx, e_idx)


def _matmul_relu(X, WT, b2):
  blk = 1000

  def mm_body(x_ref, w_ref, b_ref, o_ref):
    h = jnp.dot(x_ref[...], w_ref[...], preferred_element_type=jnp.float32)
    h = jnp.maximum(h + b_ref[...], 0.0)
    o_ref[0] = h[:, :DH]
    o_ref[1] = h[:, DH:]

  return pl.pallas_call(
      mm_body,
      grid=(N_V // blk,),
      in_specs=[
          pl.BlockSpec((blk, D), lambda i: (i, 0)),
          pl.BlockSpec((D, D), lambda i: (0, 0)),
          pl.BlockSpec((1, D), lambda i: (0, 0)),
      ],
      out_specs=pl.BlockSpec((NC, blk, DH), lambda i: (0, i, 0)),
      out_shape=jax.ShapeDtypeStruct((NC, RP, DH), jnp.float32),
  )(X, WT, b2)


def kernel(X, W, b, edge_index, drop_rate):
  pad = N_V + (jnp.arange(NNZ_PAD, dtype=jnp.int32) % (RP - N_V))
  v = jnp.concatenate([edge_index[0].astype(jnp.int32), pad])
  e = jnp.concatenate([edge_index[1].astype(jnp.int32), pad])
  v = v.reshape(NS, NCHUNK, CH)
  e = e.reshape(NS, NCHUNK, CH)

  H2 = _matmul_relu(X, W.T, b.reshape(1, D))

  _, P2 = _sc_v2v_pass(H2, v, e)

  return P2[:N_V]
